# baseline (speedup 1.0000x reference)
import jax
import jax.numpy as jnp
from jax import lax
from jax.experimental import pallas as pl
from jax.experimental.pallas import tpu as pltpu
from jax.experimental.pallas import tpu_sc as plsc

NNODE = 10000
NEDGE = 320000
NPAD = 10240
CH = 128
NCHP = 2560
EPAD = NCHP * CH
NC = 2
NS = 16
SLICE = NPAD // NS
KPT = NCHP // (NC * NS)
SEG = 20
SEGE = SEG * CH
DEGPT = NCHP // NS

f32 = jnp.float32
i32 = jnp.int32


def _rsqrt16(x):
    xi = plsc.bitcast(x, i32)
    yi = jnp.full((16,), 0x5F3759DF, i32) - lax.shift_right_logical(
        xi, jnp.ones((16,), i32))
    y = plsc.bitcast(yi, f32)
    for _ in range(3):
        y = y * (1.5 - 0.5 * x * y * y)
    return y


def _fill16(v):
    return jnp.full((16,), v, i32)


def _agg_segment(nk, nvec, src_all, dst_all, w_all, feat_hbm, acc_sh,
                 rowsA, rowsB, isA, isB, idD, sgA, sgB):
    def fill(dst_idx, k):
        for gi in range(CH // 16):
            dst_idx[pl.ds(gi * 16, 16)] = src_all[pl.ds(k * CH + gi * 16, 16)]

    def filld(k):
        for gi in range(CH // 16):
            idD[pl.ds(gi * 16, 16)] = dst_all[pl.ds(k * CH + gi * 16, 16)]

    def scale(k, rows):
        @pl.loop(0, CH)
        def _(r):
            sp = plsc.load_gather(w_all, [_fill16(k * CH + r)])
            for j in range(nvec):
                rows[r, pl.ds(j * 16, 16)] = rows[r, pl.ds(j * 16, 16)] * sp

    @pl.loop(0, nk)
    def _(k):
        fill(isA, k)
        pltpu.async_copy(feat_hbm.at[isA], rowsA, sgA).wait()
        scale(k, rowsA)
        filld(k)
        pltpu.sync_copy(rowsA, acc_sh.at[idD], add=True)


def _l1_body(src_hbm, dst_hbm, w_hbm, x_hbm,
             agg_hbm, dinv_hbm, xs_hbm,
             dst_all, w_all, src_all, idx80, dbuf,
             rowsA, rowsB, isA, isB, idD,
             deg_sh, acc_sh, sgA, sgB):
    c = lax.axis_index("c")
    s = lax.axis_index("s")
    z16 = jnp.zeros((16,), f32)
    c7 = jnp.full((16,), 7, i32)
    c127 = jnp.full((16,), 127, i32)
    nrow = NPAD // CH // NS

    @pl.loop(0, CH)
    def _(r):
        for j in range(8):
            rowsA[r, pl.ds(j * 16, 16)] = z16
            rowsB[r, pl.ds(j * 16, 16)] = z16

    for m in range(NPAD // CH // 16):
        idx80[pl.ds(m * 16, 16)] = lax.iota(i32, 16) + m * 16

    for m in range(SLICE // CH):
        pltpu.sync_copy(rowsA, acc_sh.at[pl.ds(s * SLICE + m * CH, CH), :])
    pltpu.sync_copy(rowsA.at[pl.ds(0, nrow), :],
                    deg_sh.at[pl.ds(s * nrow, nrow), :])

    plsc.subcore_barrier()

    ks0 = s * DEGPT
    for t in range(DEGPT // SEG):
        pltpu.sync_copy(dst_hbm.at[pl.ds((ks0 + t * SEG) * CH, SEGE)], dst_all)
        pltpu.sync_copy(w_hbm.at[pl.ds((ks0 + t * SEG) * CH, SEGE)], w_all)

        @pl.loop(0, SEG * (CH // 16))
        def _(g):
            d16 = dst_all[pl.ds(g * 16, 16)]
            w16 = w_all[pl.ds(g * 16, 16)]
            plsc.addupdate_scatter(
                rowsB,
                [lax.shift_right_logical(d16, c7),
                 jnp.bitwise_and(d16, c127)],
                w16)

    pltpu.sync_copy(rowsB.at[pl.ds(0, NPAD // CH), :],
                    deg_sh.at[idx80], add=True)
    plsc.subcore_barrier()

    pltpu.sync_copy(deg_sh.at[pl.ds(s * nrow, nrow), :], dbuf)
    for r in range(nrow):
        for j in range(8):
            d = dbuf[r, pl.ds(j * 16, 16)]
            dbuf[r, pl.ds(j * 16, 16)] = _rsqrt16(d + 1.0)

    @pl.when(c == 0)
    def _():
        for r in range(nrow):
            pltpu.sync_copy(dbuf.at[r],
                            dinv_hbm.at[pl.ds(s * SLICE + r * CH, CH)])

    for m in range(SLICE // CH):
        pltpu.sync_copy(x_hbm.at[pl.ds(s * SLICE + m * CH, CH), :], rowsA)

        @pl.loop(0, CH)
        def _(r):
            sp = plsc.load_gather(dbuf, [_fill16(m), _fill16(r)])
            for j in range(8):
                rowsA[r, pl.ds(j * 16, 16)] = rowsA[r, pl.ds(j * 16, 16)] * sp
        pltpu.sync_copy(rowsA, xs_hbm.at[pl.ds(s * SLICE + m * CH, CH), :])

    plsc.subcore_barrier()

    ka = c * (NCHP // NC) + s * KPT
    for t in range(KPT // SEG):
        seg0 = ka + t * SEG
        pltpu.sync_copy(src_hbm.at[pl.ds(seg0 * CH, SEGE)], src_all)
        pltpu.sync_copy(dst_hbm.at[pl.ds(seg0 * CH, SEGE)], dst_all)
        pltpu.sync_copy(w_hbm.at[pl.ds(seg0 * CH, SEGE)], w_all)
        _agg_segment(SEG, 8, src_all, dst_all, w_all, xs_hbm, acc_sh,
                     rowsA, rowsB, isA, isB, idD, sgA, sgB)

    plsc.subcore_barrier()
    pltpu.sync_copy(acc_sh.at[pl.ds(s * SLICE, SLICE), :],
                    agg_hbm.at[c, pl.ds(s * SLICE, SLICE), :])


def _sc_layer1(src, dst, w, x):
    mesh = plsc.VectorSubcoreMesh(core_axis_name="c", subcore_axis_name="s",
                                  num_cores=NC, num_subcores=NS)
    return pl.kernel(
        _l1_body,
        out_type=(jax.ShapeDtypeStruct((NC, NPAD, 128), f32),
                  jax.ShapeDtypeStruct((NPAD,), f32),
                  jax.ShapeDtypeStruct((NPAD, 128), f32)),
        mesh=mesh,
        scratch_types=[
            pltpu.VMEM((SEGE,), i32),
            pltpu.VMEM((SEGE,), f32),
            pltpu.VMEM((SEGE,), i32),
            pltpu.VMEM((NPAD // CH,), i32),
            pltpu.VMEM((NPAD // CH // NS, CH), f32),
            pltpu.VMEM((CH, 128), f32),
            pltpu.VMEM((CH, 128), f32),
            pltpu.VMEM((CH,), i32),
            pltpu.VMEM((CH,), i32),
            pltpu.VMEM((CH,), i32),
            pltpu.VMEM_SHARED((NPAD // CH, CH), f32),
            pltpu.VMEM_SHARED((NPAD, 128), f32),
            pltpu.SemaphoreType.DMA,
            pltpu.SemaphoreType.DMA,
        ],
        compiler_params=pltpu.CompilerParams(needs_layout_passes=False),
        name="gcn_sc_layer1",
    )(src, dst, w, x)


def _l2_body(src_hbm, dst_hbm, w_hbm, hws_hbm, agg_hbm,
             src_all, dst_all, w_all, rowsA, rowsB, isA, isB, idD,
             acc_sh, sgA, sgB):
    c = lax.axis_index("c")
    s = lax.axis_index("s")
    z16 = jnp.zeros((16,), f32)

    @pl.loop(0, CH)
    def _(r):
        rowsA[r, pl.ds(0, 16)] = z16
        rowsA[r, pl.ds(16, 16)] = z16
    for m in range(SLICE // CH):
        pltpu.sync_copy(rowsA, acc_sh.at[pl.ds(s * SLICE + m * CH, CH), :])
    plsc.subcore_barrier()

    ka = c * (NCHP // NC) + s * KPT
    pltpu.sync_copy(src_hbm.at[pl.ds(ka * CH, KPT * CH)], src_all)
    pltpu.sync_copy(dst_hbm.at[pl.ds(ka * CH, KPT * CH)], dst_all)
    pltpu.sync_copy(w_hbm.at[pl.ds(ka * CH, KPT * CH)], w_all)
    _agg_segment(KPT, 2, src_all, dst_all, w_all, hws_hbm, acc_sh,
                 rowsA, rowsB, isA, isB, idD, sgA, sgB)

    plsc.subcore_barrier()
    pltpu.sync_copy(acc_sh.at[pl.ds(s * SLICE, SLICE), :],
                    agg_hbm.at[c, pl.ds(s * SLICE, SLICE), :])


def _sc_layer2(src, dst, w, hws):
    mesh = plsc.VectorSubcoreMesh(core_axis_name="c", subcore_axis_name="s",
                                  num_cores=NC, num_subcores=NS)
    return pl.kernel(
        _l2_body,
        out_type=jax.ShapeDtypeStruct((NC, NPAD, 32), f32),
        mesh=mesh,
        scratch_types=[
            pltpu.VMEM((KPT * CH,), i32),
            pltpu.VMEM((KPT * CH,), i32),
            pltpu.VMEM((KPT * CH,), f32),
            pltpu.VMEM((CH, 32), f32),
            pltpu.VMEM((CH, 32), f32),
            pltpu.VMEM((CH,), i32),
            pltpu.VMEM((CH,), i32),
            pltpu.VMEM((CH,), i32),
            pltpu.VMEM_SHARED((NPAD, 32), f32),
            pltpu.SemaphoreType.DMA,
            pltpu.SemaphoreType.DMA,
        ],
        compiler_params=pltpu.CompilerParams(needs_layout_passes=False,
                                             use_tc_tiling_on_sc=False),
        name="gcn_sc_layer2",
    )(src, dst, w, hws)


BM = 200


def _tc_mid_body(a0, a1, xr, dv, w1, b1, w2, hw_out, hws_out):
    d = dv[...]
    z = (a0[...] + a1[...]) * d + xr[...] * (d * d)
    h = jnp.dot(z, w1[...], preferred_element_type=f32) + b1[...]
    h = jnp.maximum(h, 0.0)
    hw = jnp.dot(h, w2[...], preferred_element_type=f32)
    hw_out[...] = hw
    hws_out[...] = hw * d


def _tc_mid(a0, a1, x, dinv_col, W1p, b1p, W2p):
    return pl.pallas_call(
        _tc_mid_body,
        grid=(NNODE // BM,),
        in_specs=[
            pl.BlockSpec((BM, 128), lambda i: (i, 0)),
            pl.BlockSpec((BM, 128), lambda i: (i, 0)),
            pl.BlockSpec((BM, 128), lambda i: (i, 0)),
            pl.BlockSpec((BM, 1), lambda i: (i, 0)),
            pl.BlockSpec((128, 256), lambda i: (0, 0)),
            pl.BlockSpec((1, 256), lambda i: (0, 0)),
            pl.BlockSpec((256, 32), lambda i: (0, 0)),
        ],
        out_specs=[
            pl.BlockSpec((BM, 32), lambda i: (i, 0)),
            pl.BlockSpec((BM, 32), lambda i: (i, 0)),
        ],
        out_shape=[
            jax.ShapeDtypeStruct((NNODE, 32), f32),
            jax.ShapeDtypeStruct((NNODE, 32), f32),
        ],
        name="gcn_tc_mid",
    )(a0, a1, x, dinv_col, W1p, b1p, W2p)


def _tc_fin_body(q0, q1, hwr, dv, b2, out):
    d = dv[...]
    out[...] = (q0[...] + q1[...]) * d + hwr[...] * (d * d) + b2[...]


def _tc_fin(q0, q1, hw, dinv_col, b2p):
    return pl.pallas_call(
        _tc_fin_body,
        grid=(NNODE // BM,),
        in_specs=[
            pl.BlockSpec((BM, 32), lambda i: (i, 0)),
            pl.BlockSpec((BM, 32), lambda i: (i, 0)),
            pl.BlockSpec((BM, 32), lambda i: (i, 0)),
            pl.BlockSpec((BM, 1), lambda i: (i, 0)),
            pl.BlockSpec((1, 32), lambda i: (0, 0)),
        ],
        out_specs=pl.BlockSpec((BM, 32), lambda i: (i, 0)),
        out_shape=jax.ShapeDtypeStruct((NNODE, 32), f32),
        name="gcn_tc_fin",
    )(q0, q1, hw, dinv_col, b2p)


def kernel(x, edge_index, edge_weight, W1, b1, W2, b2):
    src = jnp.pad(edge_index[0], (0, EPAD - NEDGE))
    dst = jnp.pad(edge_index[1], (0, EPAD - NEDGE))
    w = jnp.pad(edge_weight, (0, EPAD - NEDGE))
    xp = jnp.pad(x, ((0, NPAD - NNODE), (0, 0)))
    agg1, dinv1d, _ = _sc_layer1(src, dst, w, xp)
    dinv_col = dinv1d[:NNODE].reshape(NNODE, 1)
    W1p = jnp.pad(W1, ((0, 0), (0, 56)))
    b1p = jnp.pad(b1, (0, 56)).reshape(1, 256)
    W2p = jnp.pad(W2, ((0, 56), (0, 12)))
    hw, hws = _tc_mid(agg1[0, :NNODE], agg1[1, :NNODE], x, dinv_col,
                      W1p, b1p, W2p)
    agg2 = _sc_layer2(src, dst, w, hws)
    b2p = jnp.pad(b2, (0, 12)).reshape(1, 32)
    out = _tc_fin(agg2[0, :NNODE], agg2[1, :NNODE], hw, dinv_col, b2p)
    return out[:, :20]


# speedup vs baseline: 1.0079x; 1.0079x over previous
import jax
import jax.numpy as jnp
from jax import lax
from jax.experimental import pallas as pl
from jax.experimental.pallas import tpu as pltpu
from jax.experimental.pallas import tpu_sc as plsc

NNODE = 10000
NEDGE = 320000
NPAD = 10240
CH = 128
NCHP = 2560
EPAD = NCHP * CH
NC = 2
NS = 16
SLICE = NPAD // NS
KPT = NCHP // (NC * NS)
SEG = 40
SEGE = SEG * CH
DEGPT = NCHP // NS

f32 = jnp.float32
i32 = jnp.int32


def _rsqrt16(x):
    xi = plsc.bitcast(x, i32)
    yi = jnp.full((16,), 0x5F3759DF, i32) - lax.shift_right_logical(
        xi, jnp.ones((16,), i32))
    y = plsc.bitcast(yi, f32)
    for _ in range(3):
        y = y * (1.5 - 0.5 * x * y * y)
    return y


def _fill16(v):
    return jnp.full((16,), v, i32)


def _agg_segment(nk, nvec, src_all, dst_all, w_all, feat_hbm, acc_sh,
                 rowsA, rowsB, isA, isB, idD, sgA, sgB):
    def fill(dst_idx, k):
        for gi in range(CH // 16):
            dst_idx[pl.ds(gi * 16, 16)] = src_all[pl.ds(k * CH + gi * 16, 16)]

    def filld(k):
        for gi in range(CH // 16):
            idD[pl.ds(gi * 16, 16)] = dst_all[pl.ds(k * CH + gi * 16, 16)]

    def scale(k, rows):
        @pl.loop(0, CH)
        def _(r):
            sp = plsc.load_gather(w_all, [_fill16(k * CH + r)])
            for j in range(nvec):
                rows[r, pl.ds(j * 16, 16)] = rows[r, pl.ds(j * 16, 16)] * sp

    @pl.loop(0, nk)
    def _(k):
        fill(isA, k)
        pltpu.async_copy(feat_hbm.at[isA], rowsA, sgA).wait()
        scale(k, rowsA)
        filld(k)
        pltpu.sync_copy(rowsA, acc_sh.at[idD], add=True)


def _l1_body(src_hbm, dst_hbm, w_hbm, x_hbm,
             agg_hbm, dinv_hbm, xs_hbm,
             dst_all, w_all, src_all, idx80, dbuf,
             rowsA, isA, isB, idD,
             deg_sh, acc_sh, sgA, sgB):
    c = lax.axis_index("c")
    s = lax.axis_index("s")
    z16 = jnp.zeros((16,), f32)
    c7 = jnp.full((16,), 7, i32)
    c127 = jnp.full((16,), 127, i32)
    nrow = NPAD // CH // NS

    @pl.loop(0, CH)
    def _(r):
        for j in range(8):
            rowsA[r, pl.ds(j * 16, 16)] = z16

    for m in range(NPAD // CH // 16):
        idx80[pl.ds(m * 16, 16)] = lax.iota(i32, 16) + m * 16

    for m in range(SLICE // CH):
        pltpu.sync_copy(rowsA, acc_sh.at[pl.ds(s * SLICE + m * CH, CH), :])
    pltpu.sync_copy(rowsA.at[pl.ds(0, nrow), :],
                    deg_sh.at[pl.ds(s * nrow, nrow), :])

    plsc.subcore_barrier()

    ks0 = s * DEGPT
    for t in range(DEGPT // SEG):
        pltpu.sync_copy(dst_hbm.at[pl.ds((ks0 + t * SEG) * CH, SEGE)], dst_all)
        pltpu.sync_copy(w_hbm.at[pl.ds((ks0 + t * SEG) * CH, SEGE)], w_all)

        @pl.loop(0, SEG * (CH // 16))
        def _(g):
            d16 = dst_all[pl.ds(g * 16, 16)]
            w16 = w_all[pl.ds(g * 16, 16)]
            plsc.addupdate_scatter(
                rowsA,
                [lax.shift_right_logical(d16, c7),
                 jnp.bitwise_and(d16, c127)],
                w16)

    pltpu.sync_copy(rowsA.at[pl.ds(0, NPAD // CH), :],
                    deg_sh.at[idx80], add=True)
    plsc.subcore_barrier()

    pltpu.sync_copy(deg_sh.at[pl.ds(s * nrow, nrow), :], dbuf)
    for r in range(nrow):
        for j in range(8):
            d = dbuf[r, pl.ds(j * 16, 16)]
            dbuf[r, pl.ds(j * 16, 16)] = _rsqrt16(d + 1.0)

    @pl.when(c == 0)
    def _():
        for r in range(nrow):
            pltpu.sync_copy(dbuf.at[r],
                            dinv_hbm.at[pl.ds(s * SLICE + r * CH, CH)])

    for m in range(SLICE // CH):
        pltpu.sync_copy(x_hbm.at[pl.ds(s * SLICE + m * CH, CH), :], rowsA)

        @pl.loop(0, CH)
        def _(r):
            sp = plsc.load_gather(dbuf, [_fill16(m), _fill16(r)])
            for j in range(8):
                rowsA[r, pl.ds(j * 16, 16)] = rowsA[r, pl.ds(j * 16, 16)] * sp
        pltpu.sync_copy(rowsA, xs_hbm.at[pl.ds(s * SLICE + m * CH, CH), :])

    plsc.subcore_barrier()

    ka = c * (NCHP // NC) + s * KPT
    for t in range(KPT // SEG):
        seg0 = ka + t * SEG
        pltpu.sync_copy(src_hbm.at[pl.ds(seg0 * CH, SEGE)], src_all)
        pltpu.sync_copy(dst_hbm.at[pl.ds(seg0 * CH, SEGE)], dst_all)
        pltpu.sync_copy(w_hbm.at[pl.ds(seg0 * CH, SEGE)], w_all)
        _agg_segment(SEG, 8, src_all, dst_all, w_all, xs_hbm, acc_sh,
                     rowsA, None, isA, isB, idD, sgA, sgB)

    plsc.subcore_barrier()
    pltpu.sync_copy(acc_sh.at[pl.ds(s * SLICE, SLICE), :],
                    agg_hbm.at[c, pl.ds(s * SLICE, SLICE), :])


def _sc_layer1(src, dst, w, x):
    mesh = plsc.VectorSubcoreMesh(core_axis_name="c", subcore_axis_name="s",
                                  num_cores=NC, num_subcores=NS)
    return pl.kernel(
        _l1_body,
        out_type=(jax.ShapeDtypeStruct((NC, NPAD, 128), f32),
                  jax.ShapeDtypeStruct((NPAD,), f32),
                  jax.ShapeDtypeStruct((NPAD, 128), f32)),
        mesh=mesh,
        scratch_types=[
            pltpu.VMEM((SEGE,), i32),
            pltpu.VMEM((SEGE,), f32),
            pltpu.VMEM((SEGE,), i32),
            pltpu.VMEM((NPAD // CH,), i32),
            pltpu.VMEM((NPAD // CH // NS, CH), f32),
            pltpu.VMEM((CH, 128), f32),
            pltpu.VMEM((CH,), i32),
            pltpu.VMEM((CH,), i32),
            pltpu.VMEM((CH,), i32),
            pltpu.VMEM_SHARED((NPAD // CH, CH), f32),
            pltpu.VMEM_SHARED((NPAD, 128), f32),
            pltpu.SemaphoreType.DMA,
            pltpu.SemaphoreType.DMA,
        ],
        compiler_params=pltpu.CompilerParams(needs_layout_passes=False),
        name="gcn_sc_layer1",
    )(src, dst, w, x)


def _l2_body(src_hbm, dst_hbm, w_hbm, hws_hbm, agg_hbm,
             src_all, dst_all, w_all, rowsA, rowsB, isA, isB, idD,
             acc_sh, sgA, sgB):
    c = lax.axis_index("c")
    s = lax.axis_index("s")
    z16 = jnp.zeros((16,), f32)

    @pl.loop(0, CH)
    def _(r):
        rowsA[r, pl.ds(0, 16)] = z16
        rowsA[r, pl.ds(16, 16)] = z16
    for m in range(SLICE // CH):
        pltpu.sync_copy(rowsA, acc_sh.at[pl.ds(s * SLICE + m * CH, CH), :])
    plsc.subcore_barrier()

    ka = c * (NCHP // NC) + s * KPT
    pltpu.sync_copy(src_hbm.at[pl.ds(ka * CH, KPT * CH)], src_all)
    pltpu.sync_copy(dst_hbm.at[pl.ds(ka * CH, KPT * CH)], dst_all)
    pltpu.sync_copy(w_hbm.at[pl.ds(ka * CH, KPT * CH)], w_all)
    _agg_segment(KPT, 2, src_all, dst_all, w_all, hws_hbm, acc_sh,
                 rowsA, rowsB, isA, isB, idD, sgA, sgB)

    plsc.subcore_barrier()
    pltpu.sync_copy(acc_sh.at[pl.ds(s * SLICE, SLICE), :],
                    agg_hbm.at[c, pl.ds(s * SLICE, SLICE), :])


def _sc_layer2(src, dst, w, hws):
    mesh = plsc.VectorSubcoreMesh(core_axis_name="c", subcore_axis_name="s",
                                  num_cores=NC, num_subcores=NS)
    return pl.kernel(
        _l2_body,
        out_type=jax.ShapeDtypeStruct((NC, NPAD, 32), f32),
        mesh=mesh,
        scratch_types=[
            pltpu.VMEM((KPT * CH,), i32),
            pltpu.VMEM((KPT * CH,), i32),
            pltpu.VMEM((KPT * CH,), f32),
            pltpu.VMEM((CH, 32), f32),
            pltpu.VMEM((CH, 32), f32),
            pltpu.VMEM((CH,), i32),
            pltpu.VMEM((CH,), i32),
            pltpu.VMEM((CH,), i32),
            pltpu.VMEM_SHARED((NPAD, 32), f32),
            pltpu.SemaphoreType.DMA,
            pltpu.SemaphoreType.DMA,
        ],
        compiler_params=pltpu.CompilerParams(needs_layout_passes=False,
                                             use_tc_tiling_on_sc=False),
        name="gcn_sc_layer2",
    )(src, dst, w, hws)


BM = 200


def _tc_mid_body(a0, a1, xr, dv, w1, b1, w2, hw_out, hws_out):
    d = dv[...]
    z = (a0[...] + a1[...]) * d + xr[...] * (d * d)
    h = jnp.dot(z, w1[...], preferred_element_type=f32) + b1[...]
    h = jnp.maximum(h, 0.0)
    hw = jnp.dot(h, w2[...], preferred_element_type=f32)
    hw_out[...] = hw
    hws_out[...] = hw * d


def _tc_mid(a0, a1, x, dinv_col, W1p, b1p, W2p):
    return pl.pallas_call(
        _tc_mid_body,
        grid=(NNODE // BM,),
        in_specs=[
            pl.BlockSpec((BM, 128), lambda i: (i, 0)),
            pl.BlockSpec((BM, 128), lambda i: (i, 0)),
            pl.BlockSpec((BM, 128), lambda i: (i, 0)),
            pl.BlockSpec((BM, 1), lambda i: (i, 0)),
            pl.BlockSpec((128, 256), lambda i: (0, 0)),
            pl.BlockSpec((1, 256), lambda i: (0, 0)),
            pl.BlockSpec((256, 32), lambda i: (0, 0)),
        ],
        out_specs=[
            pl.BlockSpec((BM, 32), lambda i: (i, 0)),
            pl.BlockSpec((BM, 32), lambda i: (i, 0)),
        ],
        out_shape=[
            jax.ShapeDtypeStruct((NNODE, 32), f32),
            jax.ShapeDtypeStruct((NNODE, 32), f32),
        ],
        name="gcn_tc_mid",
    )(a0, a1, x, dinv_col, W1p, b1p, W2p)


def _tc_fin_body(q0, q1, hwr, dv, b2, out):
    d = dv[...]
    out[...] = (q0[...] + q1[...]) * d + hwr[...] * (d * d) + b2[...]


def _tc_fin(q0, q1, hw, dinv_col, b2p):
    return pl.pallas_call(
        _tc_fin_body,
        grid=(NNODE // BM,),
        in_specs=[
            pl.BlockSpec((BM, 32), lambda i: (i, 0)),
            pl.BlockSpec((BM, 32), lambda i: (i, 0)),
            pl.BlockSpec((BM, 32), lambda i: (i, 0)),
            pl.BlockSpec((BM, 1), lambda i: (i, 0)),
            pl.BlockSpec((1, 32), lambda i: (0, 0)),
        ],
        out_specs=pl.BlockSpec((BM, 32), lambda i: (i, 0)),
        out_shape=jax.ShapeDtypeStruct((NNODE, 32), f32),
        name="gcn_tc_fin",
    )(q0, q1, hw, dinv_col, b2p)


def kernel(x, edge_index, edge_weight, W1, b1, W2, b2):
    src = jnp.pad(edge_index[0], (0, EPAD - NEDGE))
    dst = jnp.pad(edge_index[1], (0, EPAD - NEDGE))
    w = jnp.pad(edge_weight, (0, EPAD - NEDGE))
    xp = jnp.pad(x, ((0, NPAD - NNODE), (0, 0)))
    agg1, dinv1d, _ = _sc_layer1(src, dst, w, xp)
    dinv_col = dinv1d[:NNODE].reshape(NNODE, 1)
    W1p = jnp.pad(W1, ((0, 0), (0, 56)))
    b1p = jnp.pad(b1, (0, 56)).reshape(1, 256)
    W2p = jnp.pad(W2, ((0, 56), (0, 12)))
    hw, hws = _tc_mid(agg1[0, :NNODE], agg1[1, :NNODE], x, dinv_col,
                      W1p, b1p, W2p)
    agg2 = _sc_layer2(src, dst, w, hws)
    b2p = jnp.pad(b2, (0, 12)).reshape(1, 32)
    out = _tc_fin(agg2[0, :NNODE], agg2[1, :NNODE], hw, dinv_col, b2p)
    return out[:, :20]


# speedup vs baseline: 2.2283x; 2.2108x over previous
import jax
import jax.numpy as jnp
from jax import lax
from jax.experimental import pallas as pl
from jax.experimental.pallas import tpu as pltpu
from jax.experimental.pallas import tpu_sc as plsc

NNODE = 10000
NEDGE = 320000
NPAD = 10240
CH = 128
NCH = 2500
EBUF = (NCH + 40) * CH
NC = 2
NS = 16
SLICE = NPAD // NS
SEG = 32
SEGE = SEG * CH
AGG_BUF = 79 * CH

f32 = jnp.float32
i32 = jnp.int32


def _rsqrt16(x):
    xi = plsc.bitcast(x, i32)
    yi = jnp.full((16,), 0x5F3759DF, i32) - lax.shift_right_logical(
        xi, jnp.ones((16,), i32))
    y = plsc.bitcast(yi, f32)
    for _ in range(3):
        y = y * (1.5 - 0.5 * x * y * y)
    return y


def _fill16(v):
    return jnp.full((16,), v, i32)


def _agg_segment(nk, nvec, src_all, dst_all, w_all, feat_hbm, acc_sh,
                 rowsA, rowsB, isA, isB, idD, sgA, sgB):
    def fill(dst_idx, k):
        for gi in range(CH // 16):
            dst_idx[pl.ds(gi * 16, 16)] = src_all[pl.ds(k * CH + gi * 16, 16)]

    def filld(k):
        for gi in range(CH // 16):
            idD[pl.ds(gi * 16, 16)] = dst_all[pl.ds(k * CH + gi * 16, 16)]

    def scale(k, rows):
        @pl.loop(0, CH)
        def _(r):
            sp = plsc.load_gather(w_all, [_fill16(k * CH + r)])
            for j in range(nvec):
                rows[r, pl.ds(j * 16, 16)] = rows[r, pl.ds(j * 16, 16)] * sp

    npair = nk // 2

    @pl.when(nk > 0)
    def _():
        fill(isA, 0)
        pltpu.async_copy(feat_hbm.at[isA], rowsA, sgA)

    @pl.loop(0, npair)
    def _(g):
        k0 = g * 2
        pltpu.make_async_copy(feat_hbm.at[isA], rowsA, sgA).wait()
        fill(isB, k0 + 1)
        pltpu.async_copy(feat_hbm.at[isB], rowsB, sgB)
        scale(k0, rowsA)
        filld(k0)
        pltpu.sync_copy(rowsA, acc_sh.at[idD], add=True)
        pltpu.make_async_copy(feat_hbm.at[isB], rowsB, sgB).wait()

        @pl.when(k0 + 2 < nk)
        def _():
            fill(isA, k0 + 2)
            pltpu.async_copy(feat_hbm.at[isA], rowsA, sgA)
        scale(k0 + 1, rowsB)
        filld(k0 + 1)
        pltpu.sync_copy(rowsB, acc_sh.at[idD], add=True)

    @pl.when(nk - npair * 2 > 0)
    def _():
        k = npair * 2
        pltpu.make_async_copy(feat_hbm.at[isA], rowsA, sgA).wait()
        scale(k, rowsA)
        filld(k)
        pltpu.sync_copy(rowsA, acc_sh.at[idD], add=True)


def _l1_body(src_hbm, dst_hbm, w_hbm, x_hbm,
             agg_hbm, dinv_hbm, xs_hbm,
             dst_all, w_all, src_all, idx80, dbuf,
             rowsA, rowsB, isA, isB, idD,
             deg_sh, acc_sh, sgA, sgB):
    c = lax.axis_index("c")
    s = lax.axis_index("s")
    z16 = jnp.zeros((16,), f32)
    c7 = jnp.full((16,), 7, i32)
    c127 = jnp.full((16,), 127, i32)
    nrow = NPAD // CH // NS

    @pl.loop(0, CH)
    def _(r):
        for j in range(8):
            rowsA[r, pl.ds(j * 16, 16)] = z16

    for m in range(NPAD // CH // 16):
        idx80[pl.ds(m * 16, 16)] = lax.iota(i32, 16) + m * 16

    for m in range(SLICE // CH):
        pltpu.sync_copy(rowsA, acc_sh.at[pl.ds(s * SLICE + m * CH, CH), :])
    pltpu.sync_copy(rowsA.at[pl.ds(0, nrow), :],
                    deg_sh.at[pl.ds(s * nrow, nrow), :])

    plsc.subcore_barrier()

    ks0 = (s * NCH) // NS
    ks1 = ((s + 1) * NCH) // NS
    for t in range(5):
        seg0 = ks0 + t * SEG
        nk = jnp.minimum(SEG, ks1 - seg0)

        @pl.when(nk > 0)
        def _():
            pltpu.sync_copy(dst_hbm.at[pl.ds(seg0 * CH, SEGE)], dst_all)
            pltpu.sync_copy(w_hbm.at[pl.ds(seg0 * CH, SEGE)], w_all)

            @pl.loop(0, nk * (CH // 16))
            def _(g):
                d16 = dst_all[pl.ds(g * 16, 16)]
                w16 = w_all[pl.ds(g * 16, 16)]
                plsc.addupdate_scatter(
                    rowsA,
                    [lax.shift_right_logical(d16, c7),
                     jnp.bitwise_and(d16, c127)],
                    w16)

    pltpu.sync_copy(rowsA.at[pl.ds(0, NPAD // CH), :],
                    deg_sh.at[idx80], add=True)
    plsc.subcore_barrier()

    pltpu.sync_copy(deg_sh.at[pl.ds(s * nrow, nrow), :], dbuf)
    for r in range(nrow):
        for j in range(8):
            d = dbuf[r, pl.ds(j * 16, 16)]
            dbuf[r, pl.ds(j * 16, 16)] = _rsqrt16(d + 1.0)

    @pl.when(c == 0)
    def _():
        for r in range(nrow):
            pltpu.sync_copy(dbuf.at[r],
                            dinv_hbm.at[pl.ds(s * SLICE + r * CH, CH)])

    for m in range(SLICE // CH):
        pltpu.sync_copy(x_hbm.at[pl.ds(s * SLICE + m * CH, CH), :], rowsA)

        @pl.loop(0, CH)
        def _(r):
            sp = plsc.load_gather(dbuf, [_fill16(m), _fill16(r)])
            for j in range(8):
                rowsA[r, pl.ds(j * 16, 16)] = rowsA[r, pl.ds(j * 16, 16)] * sp
        pltpu.sync_copy(rowsA, xs_hbm.at[pl.ds(s * SLICE + m * CH, CH), :])

    plsc.subcore_barrier()

    ka0 = c * (NCH // NC) + (s * (NCH // NC)) // NS
    ka1 = c * (NCH // NC) + ((s + 1) * (NCH // NC)) // NS
    for t in range(3):
        seg0 = ka0 + t * SEG
        nk = jnp.minimum(SEG, ka1 - seg0)

        @pl.when(nk > 0)
        def _():
            pltpu.sync_copy(src_hbm.at[pl.ds(seg0 * CH, SEGE)], src_all)
            pltpu.sync_copy(dst_hbm.at[pl.ds(seg0 * CH, SEGE)], dst_all)
            pltpu.sync_copy(w_hbm.at[pl.ds(seg0 * CH, SEGE)], w_all)
            _agg_segment(nk, 8, src_all, dst_all, w_all, xs_hbm, acc_sh,
                         rowsA, rowsB, isA, isB, idD, sgA, sgB)

    plsc.subcore_barrier()
    pltpu.sync_copy(acc_sh.at[pl.ds(s * SLICE, SLICE), :],
                    agg_hbm.at[c, pl.ds(s * SLICE, SLICE), :])


def _sc_layer1(src, dst, w, x):
    mesh = plsc.VectorSubcoreMesh(core_axis_name="c", subcore_axis_name="s",
                                  num_cores=NC, num_subcores=NS)
    return pl.kernel(
        _l1_body,
        out_type=(jax.ShapeDtypeStruct((NC, NPAD, 128), f32),
                  jax.ShapeDtypeStruct((NPAD,), f32),
                  jax.ShapeDtypeStruct((NPAD, 128), f32)),
        mesh=mesh,
        scratch_types=[
            pltpu.VMEM((SEGE,), i32),
            pltpu.VMEM((SEGE,), f32),
            pltpu.VMEM((SEGE,), i32),
            pltpu.VMEM((NPAD // CH,), i32),
            pltpu.VMEM((NPAD // CH // NS, CH), f32),
            pltpu.VMEM((CH, 128), f32),
            pltpu.VMEM((CH, 128), f32),
            pltpu.VMEM((CH,), i32),
            pltpu.VMEM((CH,), i32),
            pltpu.VMEM((CH,), i32),
            pltpu.VMEM_SHARED((NPAD // CH, CH), f32),
            pltpu.VMEM_SHARED((NPAD, 128), f32),
            pltpu.SemaphoreType.DMA,
            pltpu.SemaphoreType.DMA,
        ],
        compiler_params=pltpu.CompilerParams(needs_layout_passes=False),
        name="gcn_sc_layer1",
    )(src, dst, w, x)


def _l2_body(src_hbm, dst_hbm, w_hbm, hws_hbm, agg_hbm,
             src_all, dst_all, w_all, rowsA, rowsB, isA, isB, idD,
             acc_sh, sgA, sgB):
    c = lax.axis_index("c")
    s = lax.axis_index("s")
    z16 = jnp.zeros((16,), f32)

    @pl.loop(0, CH)
    def _(r):
        rowsA[r, pl.ds(0, 16)] = z16
        rowsA[r, pl.ds(16, 16)] = z16
    for m in range(SLICE // CH):
        pltpu.sync_copy(rowsA, acc_sh.at[pl.ds(s * SLICE + m * CH, CH), :])
    plsc.subcore_barrier()

    ka0 = c * (NCH // NC) + (s * (NCH // NC)) // NS
    ka1 = c * (NCH // NC) + ((s + 1) * (NCH // NC)) // NS
    pltpu.sync_copy(src_hbm.at[pl.ds(ka0 * CH, AGG_BUF)], src_all)
    pltpu.sync_copy(dst_hbm.at[pl.ds(ka0 * CH, AGG_BUF)], dst_all)
    pltpu.sync_copy(w_hbm.at[pl.ds(ka0 * CH, AGG_BUF)], w_all)
    _agg_segment(ka1 - ka0, 2, src_all, dst_all, w_all, hws_hbm, acc_sh,
                 rowsA, rowsB, isA, isB, idD, sgA, sgB)

    plsc.subcore_barrier()
    pltpu.sync_copy(acc_sh.at[pl.ds(s * SLICE, SLICE), :],
                    agg_hbm.at[c, pl.ds(s * SLICE, SLICE), :])


def _sc_layer2(src, dst, w, hws):
    mesh = plsc.VectorSubcoreMesh(core_axis_name="c", subcore_axis_name="s",
                                  num_cores=NC, num_subcores=NS)
    return pl.kernel(
        _l2_body,
        out_type=jax.ShapeDtypeStruct((NC, NPAD, 32), f32),
        mesh=mesh,
        scratch_types=[
            pltpu.VMEM((AGG_BUF,), i32),
            pltpu.VMEM((AGG_BUF,), i32),
            pltpu.VMEM((AGG_BUF,), f32),
            pltpu.VMEM((CH, 32), f32),
            pltpu.VMEM((CH, 32), f32),
            pltpu.VMEM((CH,), i32),
            pltpu.VMEM((CH,), i32),
            pltpu.VMEM((CH,), i32),
            pltpu.VMEM_SHARED((NPAD, 32), f32),
            pltpu.SemaphoreType.DMA,
            pltpu.SemaphoreType.DMA,
        ],
        compiler_params=pltpu.CompilerParams(needs_layout_passes=False,
                                             use_tc_tiling_on_sc=False),
        name="gcn_sc_layer2",
    )(src, dst, w, hws)


BM = 200


def _tc_mid_body(a0, a1, xr, dv, w1, b1, w2, hw_out, hws_out):
    d = dv[...]
    z = (a0[...] + a1[...]) * d + xr[...] * (d * d)
    h = jnp.dot(z, w1[...], preferred_element_type=f32) + b1[...]
    h = jnp.maximum(h, 0.0)
    hw = jnp.dot(h, w2[...], preferred_element_type=f32)
    hw_out[...] = hw
    hws_out[...] = hw * d


def _tc_mid(a0, a1, x, dinv_col, W1p, b1p, W2p):
    return pl.pallas_call(
        _tc_mid_body,
        grid=(NNODE // BM,),
        in_specs=[
            pl.BlockSpec((BM, 128), lambda i: (i, 0)),
            pl.BlockSpec((BM, 128), lambda i: (i, 0)),
            pl.BlockSpec((BM, 128), lambda i: (i, 0)),
            pl.BlockSpec((BM, 1), lambda i: (i, 0)),
            pl.BlockSpec((128, 256), lambda i: (0, 0)),
            pl.BlockSpec((1, 256), lambda i: (0, 0)),
            pl.BlockSpec((256, 32), lambda i: (0, 0)),
        ],
        out_specs=[
            pl.BlockSpec((BM, 32), lambda i: (i, 0)),
            pl.BlockSpec((BM, 32), lambda i: (i, 0)),
        ],
        out_shape=[
            jax.ShapeDtypeStruct((NNODE, 32), f32),
            jax.ShapeDtypeStruct((NNODE, 32), f32),
        ],
        name="gcn_tc_mid",
    )(a0, a1, x, dinv_col, W1p, b1p, W2p)


def _tc_fin_body(q0, q1, hwr, dv, b2, out):
    d = dv[...]
    out[...] = (q0[...] + q1[...]) * d + hwr[...] * (d * d) + b2[...]


def _tc_fin(q0, q1, hw, dinv_col, b2p):
    return pl.pallas_call(
        _tc_fin_body,
        grid=(NNODE // BM,),
        in_specs=[
            pl.BlockSpec((BM, 32), lambda i: (i, 0)),
            pl.BlockSpec((BM, 32), lambda i: (i, 0)),
            pl.BlockSpec((BM, 32), lambda i: (i, 0)),
            pl.BlockSpec((BM, 1), lambda i: (i, 0)),
            pl.BlockSpec((1, 32), lambda i: (0, 0)),
        ],
        out_specs=pl.BlockSpec((BM, 32), lambda i: (i, 0)),
        out_shape=jax.ShapeDtypeStruct((NNODE, 32), f32),
        name="gcn_tc_fin",
    )(q0, q1, hw, dinv_col, b2p)


def kernel(x, edge_index, edge_weight, W1, b1, W2, b2):
    src = jnp.pad(edge_index[0], (0, EBUF - NEDGE))
    dst = jnp.pad(edge_index[1], (0, EBUF - NEDGE))
    w = jnp.pad(edge_weight, (0, EBUF - NEDGE))
    xp = jnp.pad(x, ((0, NPAD - NNODE), (0, 0)))
    agg1, dinv1d, _ = _sc_layer1(src, dst, w, xp)
    dinv_col = dinv1d[:NNODE].reshape(NNODE, 1)
    W1p = jnp.pad(W1, ((0, 0), (0, 56)))
    b1p = jnp.pad(b1, (0, 56)).reshape(1, 256)
    W2p = jnp.pad(W2, ((0, 56), (0, 12)))
    hw, hws = _tc_mid(agg1[0, :NNODE], agg1[1, :NNODE], x, dinv_col,
                      W1p, b1p, W2p)
    agg2 = _sc_layer2(src, dst, w, hws)
    b2p = jnp.pad(b2, (0, 12)).reshape(1, 32)
    out = _tc_fin(agg2[0, :NNODE], agg2[1, :NNODE], hw, dinv_col, b2p)
    return out[:, :20]


# speedup vs baseline: 2.2394x; 1.0050x over previous
import jax
import jax.numpy as jnp
from jax import lax
from jax.experimental import pallas as pl
from jax.experimental.pallas import tpu as pltpu
from jax.experimental.pallas import tpu_sc as plsc

NNODE = 10000
NEDGE = 320000
NPAD = 10240
CH = 128
NCH = 2500
EBUF = (NCH + 40) * CH
NC = 2
NS = 16
SLICE = NPAD // NS
SEG = 32
SEGE = SEG * CH
AGG_BUF = 79 * CH

f32 = jnp.float32
i32 = jnp.int32


def _rsqrt16(x):
    xi = plsc.bitcast(x, i32)
    yi = jnp.full((16,), 0x5F3759DF, i32) - lax.shift_right_logical(
        xi, jnp.ones((16,), i32))
    y = plsc.bitcast(yi, f32)
    for _ in range(3):
        y = y * (1.5 - 0.5 * x * y * y)
    return y


def _fill16(v):
    return jnp.full((16,), v, i32)


def _agg_segment(nk, nvec, src_all, dst_all, w_all, feat_hbm, acc_sh,
                 rowsA, rowsB, isA, isB, idD, sgA, sgB, scA, scB):
    def fill(dst_idx, k):
        for gi in range(CH // 16):
            dst_idx[pl.ds(gi * 16, 16)] = src_all[pl.ds(k * CH + gi * 16, 16)]

    def filld(k):
        for gi in range(CH // 16):
            idD[pl.ds(gi * 16, 16)] = dst_all[pl.ds(k * CH + gi * 16, 16)]

    def scale(k, rows):
        @pl.loop(0, CH)
        def _(r):
            sp = plsc.load_gather(w_all, [_fill16(k * CH + r)])
            for j in range(nvec):
                rows[r, pl.ds(j * 16, 16)] = rows[r, pl.ds(j * 16, 16)] * sp

    npair = nk // 2

    def c_wait(rows):
        pltpu.make_async_copy(rows, acc_sh.at[idD], scA).wait()

    @pl.when(nk > 0)
    def _():
        fill(isA, 0)
        pltpu.async_copy(feat_hbm.at[isA], rowsA, sgA)

    @pl.loop(0, npair)
    def _(g):
        k0 = g * 2
        pltpu.make_async_copy(feat_hbm.at[isA], rowsA, sgA).wait()

        @pl.when(g > 0)
        def _():
            pltpu.make_async_copy(rowsB, acc_sh.at[idD], scB).wait()
        fill(isB, k0 + 1)
        pltpu.async_copy(feat_hbm.at[isB], rowsB, sgB)
        scale(k0, rowsA)
        filld(k0)
        pltpu.async_copy(rowsA, acc_sh.at[idD], scA, add=True)
        pltpu.make_async_copy(feat_hbm.at[isB], rowsB, sgB).wait()

        @pl.when(k0 + 2 < nk)
        def _():
            pltpu.make_async_copy(rowsA, acc_sh.at[idD], scA).wait()
            fill(isA, k0 + 2)
            pltpu.async_copy(feat_hbm.at[isA], rowsA, sgA)
        scale(k0 + 1, rowsB)
        filld(k0 + 1)
        pltpu.async_copy(rowsB, acc_sh.at[idD], scB, add=True)

    @pl.when(nk - npair * 2 > 0)
    def _():
        k = npair * 2
        pltpu.make_async_copy(feat_hbm.at[isA], rowsA, sgA).wait()
        scale(k, rowsA)
        filld(k)
        pltpu.sync_copy(rowsA, acc_sh.at[idD], add=True)

    @pl.when((npair > 0) & (nk - npair * 2 == 0))
    def _():
        pltpu.make_async_copy(rowsA, acc_sh.at[idD], scA).wait()

    @pl.when(npair > 0)
    def _():
        pltpu.make_async_copy(rowsB, acc_sh.at[idD], scB).wait()


def _l1_body(src_hbm, dst_hbm, w_hbm, x_hbm,
             agg_hbm, dinv_hbm, xs_hbm,
             dst_all, w_all, src_all, idx80, dbuf,
             rowsA, rowsB, isA, isB, idD,
             deg_sh, acc_sh, sgA, sgB, scA, scB):
    c = lax.axis_index("c")
    s = lax.axis_index("s")
    z16 = jnp.zeros((16,), f32)
    c7 = jnp.full((16,), 7, i32)
    c127 = jnp.full((16,), 127, i32)
    nrow = NPAD // CH // NS

    @pl.loop(0, CH)
    def _(r):
        for j in range(8):
            rowsA[r, pl.ds(j * 16, 16)] = z16

    for m in range(NPAD // CH // 16):
        idx80[pl.ds(m * 16, 16)] = lax.iota(i32, 16) + m * 16

    for m in range(SLICE // CH):
        pltpu.sync_copy(rowsA, acc_sh.at[pl.ds(s * SLICE + m * CH, CH), :])
    pltpu.sync_copy(rowsA.at[pl.ds(0, nrow), :],
                    deg_sh.at[pl.ds(s * nrow, nrow), :])

    plsc.subcore_barrier()

    ks0 = (s * NCH) // NS
    ks1 = ((s + 1) * NCH) // NS
    for t in range(5):
        seg0 = ks0 + t * SEG
        nk = jnp.minimum(SEG, ks1 - seg0)

        @pl.when(nk > 0)
        def _():
            pltpu.sync_copy(dst_hbm.at[pl.ds(seg0 * CH, SEGE)], dst_all)
            pltpu.sync_copy(w_hbm.at[pl.ds(seg0 * CH, SEGE)], w_all)

            @pl.loop(0, nk * (CH // 16))
            def _(g):
                d16 = dst_all[pl.ds(g * 16, 16)]
                w16 = w_all[pl.ds(g * 16, 16)]
                plsc.addupdate_scatter(
                    rowsA,
                    [lax.shift_right_logical(d16, c7),
                     jnp.bitwise_and(d16, c127)],
                    w16)

    pltpu.sync_copy(rowsA.at[pl.ds(0, NPAD // CH), :],
                    deg_sh.at[idx80], add=True)
    plsc.subcore_barrier()

    pltpu.sync_copy(deg_sh.at[pl.ds(s * nrow, nrow), :], dbuf)
    for r in range(nrow):
        for j in range(8):
            d = dbuf[r, pl.ds(j * 16, 16)]
            dbuf[r, pl.ds(j * 16, 16)] = _rsqrt16(d + 1.0)

    @pl.when(c == 0)
    def _():
        for r in range(nrow):
            pltpu.sync_copy(dbuf.at[r],
                            dinv_hbm.at[pl.ds(s * SLICE + r * CH, CH)])

    for m in range(SLICE // CH):
        pltpu.sync_copy(x_hbm.at[pl.ds(s * SLICE + m * CH, CH), :], rowsA)

        @pl.loop(0, CH)
        def _(r):
            sp = plsc.load_gather(dbuf, [_fill16(m), _fill16(r)])
            for j in range(8):
                rowsA[r, pl.ds(j * 16, 16)] = rowsA[r, pl.ds(j * 16, 16)] * sp
        pltpu.sync_copy(rowsA, xs_hbm.at[pl.ds(s * SLICE + m * CH, CH), :])

    plsc.subcore_barrier()

    ka0 = c * (NCH // NC) + (s * (NCH // NC)) // NS
    ka1 = c * (NCH // NC) + ((s + 1) * (NCH // NC)) // NS
    for t in range(3):
        seg0 = ka0 + t * SEG
        nk = jnp.minimum(SEG, ka1 - seg0)

        @pl.when(nk > 0)
        def _():
            pltpu.sync_copy(src_hbm.at[pl.ds(seg0 * CH, SEGE)], src_all)
            pltpu.sync_copy(dst_hbm.at[pl.ds(seg0 * CH, SEGE)], dst_all)
            pltpu.sync_copy(w_hbm.at[pl.ds(seg0 * CH, SEGE)], w_all)
            _agg_segment(nk, 8, src_all, dst_all, w_all, xs_hbm, acc_sh,
                         rowsA, rowsB, isA, isB, idD, sgA, sgB, scA, scB)

    plsc.subcore_barrier()
    pltpu.sync_copy(acc_sh.at[pl.ds(s * SLICE, SLICE), :],
                    agg_hbm.at[c, pl.ds(s * SLICE, SLICE), :])


def _sc_layer1(src, dst, w, x):
    mesh = plsc.VectorSubcoreMesh(core_axis_name="c", subcore_axis_name="s",
                                  num_cores=NC, num_subcores=NS)
    return pl.kernel(
        _l1_body,
        out_type=(jax.ShapeDtypeStruct((NC, NPAD, 128), f32),
                  jax.ShapeDtypeStruct((NPAD,), f32),
                  jax.ShapeDtypeStruct((NPAD, 128), f32)),
        mesh=mesh,
        scratch_types=[
            pltpu.VMEM((SEGE,), i32),
            pltpu.VMEM((SEGE,), f32),
            pltpu.VMEM((SEGE,), i32),
            pltpu.VMEM((NPAD // CH,), i32),
            pltpu.VMEM((NPAD // CH // NS, CH), f32),
            pltpu.VMEM((CH, 128), f32),
            pltpu.VMEM((CH, 128), f32),
            pltpu.VMEM((CH,), i32),
            pltpu.VMEM((CH,), i32),
            pltpu.VMEM((CH,), i32),
            pltpu.VMEM_SHARED((NPAD // CH, CH), f32),
            pltpu.VMEM_SHARED((NPAD, 128), f32),
            pltpu.SemaphoreType.DMA,
            pltpu.SemaphoreType.DMA,
            pltpu.SemaphoreType.DMA,
            pltpu.SemaphoreType.DMA,
        ],
        compiler_params=pltpu.CompilerParams(needs_layout_passes=False),
        name="gcn_sc_layer1",
    )(src, dst, w, x)


def _l2_body(src_hbm, dst_hbm, w_hbm, hws_hbm, agg_hbm,
             src_all, dst_all, w_all, rowsA, rowsB, isA, isB, idD,
             acc_sh, sgA, sgB, scA, scB):
    c = lax.axis_index("c")
    s = lax.axis_index("s")
    z16 = jnp.zeros((16,), f32)

    @pl.loop(0, CH)
    def _(r):
        rowsA[r, pl.ds(0, 16)] = z16
        rowsA[r, pl.ds(16, 16)] = z16
    for m in range(SLICE // CH):
        pltpu.sync_copy(rowsA, acc_sh.at[pl.ds(s * SLICE + m * CH, CH), :])
    plsc.subcore_barrier()

    ka0 = c * (NCH // NC) + (s * (NCH // NC)) // NS
    ka1 = c * (NCH // NC) + ((s + 1) * (NCH // NC)) // NS
    pltpu.sync_copy(src_hbm.at[pl.ds(ka0 * CH, AGG_BUF)], src_all)
    pltpu.sync_copy(dst_hbm.at[pl.ds(ka0 * CH, AGG_BUF)], dst_all)
    pltpu.sync_copy(w_hbm.at[pl.ds(ka0 * CH, AGG_BUF)], w_all)
    _agg_segment(ka1 - ka0, 2, src_all, dst_all, w_all, hws_hbm, acc_sh,
                 rowsA, rowsB, isA, isB, idD, sgA, sgB, scA, scB)

    plsc.subcore_barrier()
    pltpu.sync_copy(acc_sh.at[pl.ds(s * SLICE, SLICE), :],
                    agg_hbm.at[c, pl.ds(s * SLICE, SLICE), :])


def _sc_layer2(src, dst, w, hws):
    mesh = plsc.VectorSubcoreMesh(core_axis_name="c", subcore_axis_name="s",
                                  num_cores=NC, num_subcores=NS)
    return pl.kernel(
        _l2_body,
        out_type=jax.ShapeDtypeStruct((NC, NPAD, 32), f32),
        mesh=mesh,
        scratch_types=[
            pltpu.VMEM((AGG_BUF,), i32),
            pltpu.VMEM((AGG_BUF,), i32),
            pltpu.VMEM((AGG_BUF,), f32),
            pltpu.VMEM((CH, 32), f32),
            pltpu.VMEM((CH, 32), f32),
            pltpu.VMEM((CH,), i32),
            pltpu.VMEM((CH,), i32),
            pltpu.VMEM((CH,), i32),
            pltpu.VMEM_SHARED((NPAD, 32), f32),
            pltpu.SemaphoreType.DMA,
            pltpu.SemaphoreType.DMA,
            pltpu.SemaphoreType.DMA,
            pltpu.SemaphoreType.DMA,
        ],
        compiler_params=pltpu.CompilerParams(needs_layout_passes=False,
                                             use_tc_tiling_on_sc=False),
        name="gcn_sc_layer2",
    )(src, dst, w, hws)


BM = 200


def _tc_mid_body(a0, a1, xr, dv, w1, b1, w2, hw_out, hws_out):
    d = dv[...]
    z = (a0[...] + a1[...]) * d + xr[...] * (d * d)
    h = jnp.dot(z, w1[...], preferred_element_type=f32) + b1[...]
    h = jnp.maximum(h, 0.0)
    hw = jnp.dot(h, w2[...], preferred_element_type=f32)
    hw_out[...] = hw
    hws_out[...] = hw * d


def _tc_mid(a0, a1, x, dinv_col, W1p, b1p, W2p):
    return pl.pallas_call(
        _tc_mid_body,
        grid=(NNODE // BM,),
        in_specs=[
            pl.BlockSpec((BM, 128), lambda i: (i, 0)),
            pl.BlockSpec((BM, 128), lambda i: (i, 0)),
            pl.BlockSpec((BM, 128), lambda i: (i, 0)),
            pl.BlockSpec((BM, 1), lambda i: (i, 0)),
            pl.BlockSpec((128, 256), lambda i: (0, 0)),
            pl.BlockSpec((1, 256), lambda i: (0, 0)),
            pl.BlockSpec((256, 32), lambda i: (0, 0)),
        ],
        out_specs=[
            pl.BlockSpec((BM, 32), lambda i: (i, 0)),
            pl.BlockSpec((BM, 32), lambda i: (i, 0)),
        ],
        out_shape=[
            jax.ShapeDtypeStruct((NNODE, 32), f32),
            jax.ShapeDtypeStruct((NNODE, 32), f32),
        ],
        name="gcn_tc_mid",
    )(a0, a1, x, dinv_col, W1p, b1p, W2p)


def _tc_fin_body(q0, q1, hwr, dv, b2, out):
    d = dv[...]
    out[...] = (q0[...] + q1[...]) * d + hwr[...] * (d * d) + b2[...]


def _tc_fin(q0, q1, hw, dinv_col, b2p):
    return pl.pallas_call(
        _tc_fin_body,
        grid=(NNODE // BM,),
        in_specs=[
            pl.BlockSpec((BM, 32), lambda i: (i, 0)),
            pl.BlockSpec((BM, 32), lambda i: (i, 0)),
            pl.BlockSpec((BM, 32), lambda i: (i, 0)),
            pl.BlockSpec((BM, 1), lambda i: (i, 0)),
            pl.BlockSpec((1, 32), lambda i: (0, 0)),
        ],
        out_specs=pl.BlockSpec((BM, 32), lambda i: (i, 0)),
        out_shape=jax.ShapeDtypeStruct((NNODE, 32), f32),
        name="gcn_tc_fin",
    )(q0, q1, hw, dinv_col, b2p)


def kernel(x, edge_index, edge_weight, W1, b1, W2, b2):
    src = jnp.pad(edge_index[0], (0, EBUF - NEDGE))
    dst = jnp.pad(edge_index[1], (0, EBUF - NEDGE))
    w = jnp.pad(edge_weight, (0, EBUF - NEDGE))
    xp = jnp.pad(x, ((0, NPAD - NNODE), (0, 0)))
    agg1, dinv1d, _ = _sc_layer1(src, dst, w, xp)
    dinv_col = dinv1d[:NNODE].reshape(NNODE, 1)
    W1p = jnp.pad(W1, ((0, 0), (0, 56)))
    b1p = jnp.pad(b1, (0, 56)).reshape(1, 256)
    W2p = jnp.pad(W2, ((0, 56), (0, 12)))
    hw, hws = _tc_mid(agg1[0, :NNODE], agg1[1, :NNODE], x, dinv_col,
                      W1p, b1p, W2p)
    agg2 = _sc_layer2(src, dst, w, hws)
    b2p = jnp.pad(b2, (0, 12)).reshape(1, 32)
    out = _tc_fin(agg2[0, :NNODE], agg2[1, :NNODE], hw, dinv_col, b2p)
    return out[:, :20]
